# baseline (speedup 1.0000x reference)
import functools

import jax
import jax.numpy as jnp
from jax import lax
from jax.experimental import pallas as pl
from jax.experimental.pallas import tpu as pltpu
from jax.experimental.pallas import tpu_sc as plsc

N = 320000
D = 128
S = 10000
NC, NS = 2, 16
NW = NC * NS
RPW = N // NW
C = 80
NCHUNK = RPW // C
NK = D // 16
SEG_STRIDE = 624
SEG_SPAN = 640
ZROWS = 8
NBUF = 4
NROUND = (NCHUNK - 1) // NBUF

_GDN = lax.GatherDimensionNumbers(
    offset_dims=(), collapsed_slice_dims=(0,), start_index_map=(0,))


def _lane_shuffle(v, idx):
    return lax.gather(v, idx, _GDN, (1,),
                      mode=lax.GatherScatterMode.PROMISE_IN_BOUNDS)


def _sc_body(x_hbm, wb_hbm, ids_hbm, out_hbm, shared, xv, iv, zv, wbv,
             semx, semi, semsc):
    cid = lax.axis_index("c")
    sid = lax.axis_index("s")
    wid = cid * NS + sid
    base = wid * RPW

    def issue_loads(ci, b):
        rbase = base + ci * C
        pltpu.async_copy(x_hbm.at[pl.ds(rbase, C)], xv.at[b], semx.at[b])
        pltpu.async_copy(ids_hbm.at[pl.ds(rbase, C)], iv.at[b], semi.at[b])

    def wait_loads(ci, b):
        rbase = base + ci * C
        pltpu.make_async_copy(
            x_hbm.at[pl.ds(rbase, C)], xv.at[b], semx.at[b]).wait()
        pltpu.make_async_copy(
            ids_hbm.at[pl.ds(rbase, C)], iv.at[b], semi.at[b]).wait()

    pltpu.sync_copy(wb_hbm, wbv)

    def zrow(i, _):
        zv[i // 8, pl.ds((i % 8) * 16, 16)] = jnp.zeros((16,), jnp.float32)
        return 0
    lax.fori_loop(0, ZROWS * 8, zrow, 0)
    for j in range(SEG_SPAN // ZROWS):
        pltpu.sync_copy(
            zv, shared.at[pl.ds(sid * SEG_STRIDE + j * ZROWS, ZROWS)])
    plsc.subcore_barrier()

    def weigh_and_scatter(b):
        wk = [wbv[pl.ds(k * 16, 16)] for k in range(NK)]
        bias = wbv[pl.ds(D, 16)]
        lanes = lax.iota(jnp.int32, 16)
        bfly = [(lanes ^ s).reshape(16, 1) for s in (8, 4, 2, 1)]

        @plsc.parallel_loop(0, C, unroll=6)
        def _row(r):
            xk = [xv[b, r, pl.ds(k * 16, 16)] for k in range(NK)]
            acc = xk[0] * wk[0]
            for k in range(1, NK):
                acc = acc + xk[k] * wk[k]
            for bf in bfly:
                acc = acc + _lane_shuffle(acc, bf)
            sv = 1.0 / (1.0 + jnp.exp(-(acc + bias)))
            for k in range(NK):
                xv[b, r, pl.ds(k * 16, 16)] = xk[k] * sv
        pltpu.async_copy(xv.at[b], shared.at[iv.at[b]], semsc.at[b],
                         add=True)

    def wait_scatter(b):
        pltpu.make_async_copy(
            xv.at[b], shared.at[iv.at[b]], semsc.at[b]).wait()

    for b in range(NBUF - 1):
        issue_loads(b, b)

    def piperound(g, _):
        for b in range(NBUF):
            ci = g * NBUF + b
            wait_loads(ci, b)
            bn = (b + NBUF - 1) % NBUF
            nci = ci + NBUF - 1
            if b == 0:
                @pl.when(g > 0)
                def _():
                    wait_scatter(bn)
                issue_loads(nci, bn)
            elif b == 1:
                wait_scatter(bn)
                issue_loads(nci, bn)
            else:
                @pl.when(nci < NCHUNK)
                def _():
                    wait_scatter(bn)
                    issue_loads(nci, bn)
            weigh_and_scatter(b)
        return 0
    lax.fori_loop(0, NROUND, piperound, 0)
    last = NCHUNK - 1
    lastb = last % NBUF
    wait_loads(last, lastb)
    weigh_and_scatter(lastb)
    for b in range(NBUF):
        if b != lastb:
            wait_scatter(b)
    wait_scatter(lastb)
    plsc.subcore_barrier()

    pltpu.sync_copy(
        shared.at[pl.ds(sid * SEG_STRIDE, SEG_SPAN)],
        out_hbm.at[pl.ds(cid * S + sid * SEG_STRIDE, SEG_SPAN)])


_sc_kernel = functools.partial(
    pl.kernel,
    out_type=jax.ShapeDtypeStruct((NC * S, D), jnp.float32),
    mesh=plsc.VectorSubcoreMesh(
        core_axis_name="c", subcore_axis_name="s",
        num_cores=NC, num_subcores=NS),
    scratch_types=[
        pltpu.VMEM_SHARED((S, D), jnp.float32),
        pltpu.VMEM((NBUF, C, D), jnp.float32),
        pltpu.VMEM((NBUF, C), jnp.int32),
        pltpu.VMEM((ZROWS, D), jnp.float32),
        pltpu.VMEM((D + 16,), jnp.float32),
        pltpu.SemaphoreType.DMA((NBUF,)),
        pltpu.SemaphoreType.DMA((NBUF,)),
        pltpu.SemaphoreType.DMA((NBUF,)),
    ],
)(_sc_body)


def kernel(x, segment_ids, W, b):
    ids = segment_ids.astype(jnp.int32)
    wb = jnp.concatenate(
        [W.reshape(D), jnp.broadcast_to(b.reshape(1), (16,))])
    parts = _sc_kernel(x, wb, ids)
    return parts[:S] + parts[S:]


# speedup vs baseline: 1.2709x; 1.2709x over previous
import functools

import jax
import jax.numpy as jnp
from jax import lax
from jax.experimental import pallas as pl
from jax.experimental.pallas import tpu as pltpu
from jax.experimental.pallas import tpu_sc as plsc

N = 320000
D = 128
S = 10000
NC, NS = 2, 16
NW = NC * NS
RPW = N // NW
C = 80
NCHUNK = RPW // C
NK = D // 16
SEG_STRIDE = 624
SEG_SPAN = 640
ZROWS = 40
NBUF = 4
NROUND = (NCHUNK - 1) // NBUF

_GDN = lax.GatherDimensionNumbers(
    offset_dims=(), collapsed_slice_dims=(0,), start_index_map=(0,))


def _lane_shuffle(v, idx):
    return lax.gather(v, idx, _GDN, (1,),
                      mode=lax.GatherScatterMode.PROMISE_IN_BOUNDS)


def _sc_body(x_hbm, wb_hbm, ids_hbm, out_hbm, shared, xv, iv, zv, wbv,
             semx, semi, semsc):
    cid = lax.axis_index("c")
    sid = lax.axis_index("s")
    wid = cid * NS + sid
    base = wid * RPW

    def issue_loads(ci, b):
        rbase = base + ci * C
        pltpu.async_copy(x_hbm.at[pl.ds(rbase, C)], xv.at[b], semx.at[b])
        pltpu.async_copy(ids_hbm.at[pl.ds(rbase, C)], iv.at[b], semi.at[b])

    def wait_loads(ci, b):
        rbase = base + ci * C
        pltpu.make_async_copy(
            x_hbm.at[pl.ds(rbase, C)], xv.at[b], semx.at[b]).wait()
        pltpu.make_async_copy(
            ids_hbm.at[pl.ds(rbase, C)], iv.at[b], semi.at[b]).wait()

    pltpu.sync_copy(wb_hbm, wbv)

    def zrow(i, _):
        zv[i // 8, pl.ds((i % 8) * 16, 16)] = jnp.zeros((16,), jnp.float32)
        return 0
    lax.fori_loop(0, ZROWS * 8, zrow, 0)
    for j in range(SEG_SPAN // ZROWS):
        pltpu.sync_copy(
            zv, shared.at[pl.ds(sid * SEG_STRIDE + j * ZROWS, ZROWS)])
    plsc.subcore_barrier()

    def weigh_and_scatter(b):
        wk = [wbv[pl.ds(k * 16, 16)] for k in range(NK)]
        bias = wbv[pl.ds(D, 16)]
        lanes = lax.iota(jnp.int32, 16)
        bfly = [(lanes ^ s).reshape(16, 1) for s in (8, 4, 2, 1)]

        @plsc.parallel_loop(0, C, unroll=4)
        def _row(r):
            xk = [xv[b, r, pl.ds(k * 16, 16)] for k in range(NK)]
            acc = xk[0] * wk[0]
            for k in range(1, NK):
                acc = acc + xk[k] * wk[k]
            for bf in bfly:
                acc = acc + _lane_shuffle(acc, bf)
            sv = 1.0 / (1.0 + jnp.exp(-(acc + bias)))
            for k in range(NK):
                xv[b, r, pl.ds(k * 16, 16)] = xk[k] * sv
        pltpu.async_copy(xv.at[b], shared.at[iv.at[b]], semsc.at[b],
                         add=True)

    def wait_scatter(b):
        pltpu.make_async_copy(
            xv.at[b], shared.at[iv.at[b]], semsc.at[b]).wait()

    for b in range(NBUF - 1):
        issue_loads(b, b)

    def piperound(g, _):
        for b in range(NBUF):
            ci = g * NBUF + b
            wait_loads(ci, b)
            bn = (b + NBUF - 1) % NBUF
            nci = ci + NBUF - 1
            if b == 0:
                @pl.when(g > 0)
                def _():
                    wait_scatter(bn)
                issue_loads(nci, bn)
            elif b == 1:
                wait_scatter(bn)
                issue_loads(nci, bn)
            else:
                @pl.when(nci < NCHUNK)
                def _():
                    wait_scatter(bn)
                    issue_loads(nci, bn)
            weigh_and_scatter(b)
        return 0
    lax.fori_loop(0, NROUND, piperound, 0)
    last = NCHUNK - 1
    lastb = last % NBUF
    wait_loads(last, lastb)
    weigh_and_scatter(lastb)
    for b in range(NBUF):
        if b != lastb:
            wait_scatter(b)
    wait_scatter(lastb)
    plsc.subcore_barrier()

    pltpu.sync_copy(
        shared.at[pl.ds(sid * SEG_STRIDE, SEG_SPAN)],
        out_hbm.at[pl.ds(cid * S + sid * SEG_STRIDE, SEG_SPAN)])


_sc_kernel = functools.partial(
    pl.kernel,
    out_type=jax.ShapeDtypeStruct((NC * S, D), jnp.float32),
    mesh=plsc.VectorSubcoreMesh(
        core_axis_name="c", subcore_axis_name="s",
        num_cores=NC, num_subcores=NS),
    scratch_types=[
        pltpu.VMEM_SHARED((S, D), jnp.float32),
        pltpu.VMEM((NBUF, C, D), jnp.float32),
        pltpu.VMEM((NBUF, C), jnp.int32),
        pltpu.VMEM((ZROWS, D), jnp.float32),
        pltpu.VMEM((D + 16,), jnp.float32),
        pltpu.SemaphoreType.DMA((NBUF,)),
        pltpu.SemaphoreType.DMA((NBUF,)),
        pltpu.SemaphoreType.DMA((NBUF,)),
    ],
)(_sc_body)


def kernel(x, segment_ids, W, b):
    ids = segment_ids.astype(jnp.int32)
    wb = jnp.concatenate(
        [W.reshape(D), jnp.broadcast_to(b.reshape(1), (16,))])
    parts = _sc_kernel(x, wb, ids)
    return parts[:S] + parts[S:]
